# Initial kernel scaffold; baseline (speedup 1.0000x reference)
import jax
import jax.numpy as jnp
from jax.experimental import pallas as pl


def kernel(action, table):
    raise NotImplementedError("write your pallas kernel here")


# speedup vs baseline: 4.5508x; 4.5508x over previous
import functools

import jax
import jax.numpy as jnp
from jax import lax
from jax.experimental import pallas as pl
from jax.experimental.pallas import tpu as pltpu
from jax.experimental.pallas import tpu_sc as plsc

NUM_ACTIONS = 100000
EMBED_DIM = 64
BATCH = 4096
HIST = 50

_B = BATCH * HIST
_NW = 32
_PER_W = _B // _NW
_CHUNK = 800
_STEPS = _PER_W // _CHUNK


def _gather_kernel(table_hbm, idx_hbm, out_hbm, idx_v, rows_v, sem):
    wid = lax.axis_index("s") * 2 + lax.axis_index("c")
    base = wid * _PER_W

    def body(i, carry):
        off = pl.multiple_of(base + i * _CHUNK, 8)
        pltpu.sync_copy(idx_hbm.at[pl.ds(off, _CHUNK)], idx_v)
        pltpu.async_copy(table_hbm.at[idx_v], rows_v, sem).wait()
        pltpu.sync_copy(rows_v, out_hbm.at[pl.ds(off, _CHUNK)])
        return carry

    lax.fori_loop(0, _STEPS, body, 0)


@jax.jit
def kernel(action, table):
    idx = action.reshape(_B).astype(jnp.int32)
    mesh = plsc.VectorSubcoreMesh(core_axis_name="c", subcore_axis_name="s")
    out = pl.kernel(
        _gather_kernel,
        out_type=jax.ShapeDtypeStruct((_B, EMBED_DIM), jnp.float32),
        mesh=mesh,
        scratch_types=[
            pltpu.VMEM((_CHUNK,), jnp.int32),
            pltpu.VMEM((_CHUNK, EMBED_DIM), jnp.float32),
            pltpu.SemaphoreType.DMA,
        ],
        compiler_params=pltpu.CompilerParams(use_tc_tiling_on_sc=False),
    )(table, idx)
    return out.reshape(BATCH, HIST, EMBED_DIM)


# speedup vs baseline: 4.6247x; 1.0162x over previous
import jax
import jax.numpy as jnp
from jax import lax
from jax.experimental import pallas as pl
from jax.experimental.pallas import tpu as pltpu
from jax.experimental.pallas import tpu_sc as plsc

NUM_ACTIONS = 100000
EMBED_DIM = 64
BATCH = 4096
HIST = 50

_B = BATCH * HIST
_NW = 32
_PER_W = _B // _NW
_CHUNK = 400
_STEPS = _PER_W // _CHUNK
_NBUF = 4
_PF = 2


def _gather_kernel(table_hbm, idx_hbm, out_hbm, idx_all, *scratch):
    rows = scratch[:_NBUF]
    gsem = scratch[_NBUF:2 * _NBUF]
    wsem = scratch[2 * _NBUF:3 * _NBUF]
    wid = lax.axis_index("s") * 2 + lax.axis_index("c")
    base = wid * _PER_W

    pltpu.sync_copy(idx_hbm.at[pl.ds(base, _PER_W)], idx_all)

    def start_gather(i):
        b = i % _NBUF
        return pltpu.async_copy(
            table_hbm.at[idx_all.at[pl.ds(i * _CHUNK, _CHUNK)]],
            rows[b], gsem[b])

    def start_writeback(i):
        b = i % _NBUF
        return pltpu.async_copy(
            rows[b], out_hbm.at[pl.ds(base + i * _CHUNK, _CHUNK)], wsem[b])

    gathers = {}
    writebacks = {}
    for i in range(_PF):
        gathers[i] = start_gather(i)
    for i in range(_STEPS):
        gathers.pop(i).wait()
        writebacks[i] = start_writeback(i)
        nxt = i + _PF
        if nxt < _STEPS:
            prev = nxt - _NBUF
            if prev >= 0:
                writebacks.pop(prev).wait()
            gathers[nxt] = start_gather(nxt)
    for i in sorted(writebacks):
        writebacks.pop(i).wait()


@jax.jit
def kernel(action, table):
    idx = action.reshape(_B).astype(jnp.int32)
    mesh = plsc.VectorSubcoreMesh(core_axis_name="c", subcore_axis_name="s")
    scratch = [pltpu.VMEM((_PER_W,), jnp.int32)]
    scratch += [pltpu.VMEM((_CHUNK, EMBED_DIM), jnp.float32)
                for _ in range(_NBUF)]
    scratch += [pltpu.SemaphoreType.DMA for _ in range(2 * _NBUF)]
    out = pl.kernel(
        _gather_kernel,
        out_type=jax.ShapeDtypeStruct((_B, EMBED_DIM), jnp.float32),
        mesh=mesh,
        scratch_types=scratch,
        compiler_params=pltpu.CompilerParams(use_tc_tiling_on_sc=False),
    )(table, idx)
    return out.reshape(BATCH, HIST, EMBED_DIM)
